# Initial kernel scaffold; baseline (speedup 1.0000x reference)
import jax
import jax.numpy as jnp
from jax.experimental import pallas as pl


def kernel(x, y, B):
    raise NotImplementedError("write your pallas kernel here")


# speedup vs baseline: 1.3020x; 1.3020x over previous
import functools

import jax
import jax.numpy as jnp
from jax import lax
from jax.experimental import pallas as pl
from jax.experimental.pallas import tpu as pltpu
from jax.experimental.pallas import tpu_sc as plsc

_N = 4096
_ROW_BLK = 512
_NC = 2
_NS = 16
_NW = _NC * _NS
_LANES = 16


def _colstats_body(b_ref, m_out, d_out, m_acc, d_acc):
    i = pl.program_id(0)
    blk = b_ref[...]
    bm = jnp.max(blk, axis=0, keepdims=True)

    @pl.when(i == 0)
    def _():
        m_acc[...] = jnp.full_like(bm, -jnp.inf)
        d_acc[...] = jnp.zeros_like(bm)

    m_old = m_acc[...]
    m_new = jnp.maximum(m_old, bm)
    e = jnp.where(blk != 0.0, jnp.exp(blk - m_new), 0.0)
    d_acc[...] = d_acc[...] * jnp.exp(m_old - m_new) + jnp.sum(
        e, axis=0, keepdims=True)
    m_acc[...] = m_new

    @pl.when(i == pl.num_programs(0) - 1)
    def _():
        m_out[...] = m_acc[...]
        d_out[...] = d_acc[...]


def _colstats(b):
    n = b.shape[0]
    grid = (n // _ROW_BLK,)
    m, d = pl.pallas_call(
        _colstats_body,
        grid=grid,
        in_specs=[pl.BlockSpec((_ROW_BLK, n), lambda i: (i, 0))],
        out_specs=[
            pl.BlockSpec((1, n), lambda i: (0, 0)),
            pl.BlockSpec((1, n), lambda i: (0, 0)),
        ],
        out_shape=[
            jax.ShapeDtypeStruct((1, n), jnp.float32),
            jax.ShapeDtypeStruct((1, n), jnp.float32),
        ],
        scratch_shapes=[
            pltpu.VMEM((1, n), jnp.float32),
            pltpu.VMEM((1, n), jnp.float32),
        ],
        compiler_params=pltpu.CompilerParams(
            dimension_semantics=("arbitrary",)),
    )(b)
    return m.reshape(n), d.reshape(n)


def _make_sc_gather(batch, n):
    per_w = batch // _NW
    chunk = 128
    n_chunks = per_w // chunk
    mesh = plsc.VectorSubcoreMesh(core_axis_name="c", subcore_axis_name="s")

    @functools.partial(
        pl.kernel,
        mesh=mesh,
        out_type=jax.ShapeDtypeStruct((batch,), jnp.float32),
        scratch_types=[
            pltpu.VMEM((n_chunks, chunk), jnp.int32),
            pltpu.VMEM((n_chunks, chunk), jnp.int32),
            pltpu.VMEM((n_chunks, chunk), jnp.int32),
            pltpu.VMEM((n_chunks, chunk), jnp.float32),
            pltpu.VMEM((n_chunks, chunk), jnp.float32),
            pltpu.VMEM((n_chunks, chunk), jnp.float32),
            pltpu.VMEM((n_chunks, chunk), jnp.float32),
            pltpu.SemaphoreType.DMA,
        ],
    )
    def sc_gather(x_hbm, y_hbm, bflat_hbm, m_hbm, d_hbm, out_hbm,
                  xv, yv, idxv, bv, mv, dv, outv, sem):
        wid = lax.axis_index("s") * _NC + lax.axis_index("c")
        base = wid * per_w
        for k in range(n_chunks):
            pltpu.sync_copy(x_hbm.at[pl.ds(base + k * chunk, chunk)], xv.at[k])
            pltpu.sync_copy(y_hbm.at[pl.ds(base + k * chunk, chunk)], yv.at[k])
        for k in range(n_chunks):
            for j in range(chunk // _LANES):
                sl = pl.ds(j * _LANES, _LANES)
                idxv[k, sl] = xv[k, sl] * n + yv[k, sl]
        copies = []
        for k in range(n_chunks):
            copies.append(
                pltpu.async_copy(bflat_hbm.at[idxv.at[k]], bv.at[k], sem))
            copies.append(
                pltpu.async_copy(m_hbm.at[yv.at[k]], mv.at[k], sem))
            copies.append(
                pltpu.async_copy(d_hbm.at[yv.at[k]], dv.at[k], sem))
        for c in copies:
            c.wait()
        for k in range(n_chunks):
            for j in range(chunk // _LANES):
                sl = pl.ds(j * _LANES, _LANES)
                b = bv[k, sl]
                val = jnp.exp(b - mv[k, sl]) / dv[k, sl]
                outv[k, sl] = jnp.where(b != 0.0, val, 0.0)
        for k in range(n_chunks):
            pltpu.sync_copy(outv.at[k],
                            out_hbm.at[pl.ds(base + k * chunk, chunk)])

    return sc_gather


def kernel(x, y, B):
    n = B.shape[0]
    batch = x.shape[0]
    xf = x.reshape(batch)
    yf = y.reshape(batch)
    m, d = _colstats(B)
    out = _make_sc_gather(batch, n)(xf, yf, B.reshape(n * n), m, d)
    return out.reshape(x.shape)


# speedup vs baseline: 2.4604x; 1.8897x over previous
import functools

import jax
import jax.numpy as jnp
from jax import lax
from jax.experimental import pallas as pl
from jax.experimental.pallas import tpu as pltpu
from jax.experimental.pallas import tpu_sc as plsc

_ROW_BLK = 512
_NC = 2
_NS = 16
_NW = _NC * _NS
_LANES = 16


def _colstats_body(b_ref, g_out, m_scr, d_scr):
    i = pl.program_id(0)
    blk = b_ref[...]
    bm = jnp.max(blk, axis=0, keepdims=True)
    e = jnp.where(blk != 0.0, jnp.exp(blk - bm), 0.0)
    bs = jnp.sum(e, axis=0, keepdims=True)
    m_scr[pl.ds(i, 1), :] = bm
    d_scr[pl.ds(i, 1), :] = bs

    @pl.when(i == pl.num_programs(0) - 1)
    def _():
        ms = m_scr[...]
        ds = d_scr[...]
        m = jnp.max(ms, axis=0, keepdims=True)
        d = jnp.sum(ds * jnp.exp(ms - m), axis=0, keepdims=True)
        g_out[...] = m + jnp.log(d)


def _colstats(b):
    n = b.shape[0]
    nblk = n // _ROW_BLK
    g = pl.pallas_call(
        _colstats_body,
        grid=(nblk,),
        in_specs=[pl.BlockSpec((_ROW_BLK, n), lambda i: (i, 0))],
        out_specs=pl.BlockSpec((1, n), lambda i: (0, 0)),
        out_shape=jax.ShapeDtypeStruct((1, n), jnp.float32),
        scratch_shapes=[
            pltpu.VMEM((nblk, n), jnp.float32),
            pltpu.VMEM((nblk, n), jnp.float32),
        ],
        compiler_params=pltpu.CompilerParams(
            dimension_semantics=("parallel",)),
    )(b)
    return g.reshape(n)


def _make_sc_bgather(batch, n):
    per_w = batch // _NW
    chunk = 128
    n_chunks = per_w // chunk
    mesh = plsc.VectorSubcoreMesh(core_axis_name="c", subcore_axis_name="s")

    @functools.partial(
        pl.kernel,
        mesh=mesh,
        out_type=jax.ShapeDtypeStruct((batch,), jnp.float32),
        scratch_types=[
            pltpu.VMEM((per_w,), jnp.int32),
            pltpu.VMEM((per_w,), jnp.int32),
            pltpu.VMEM((n_chunks, chunk), jnp.int32),
            pltpu.VMEM((n_chunks, chunk), jnp.float32),
            pltpu.SemaphoreType.DMA,
        ],
    )
    def sc_bgather(x_hbm, y_hbm, bflat_hbm, out_hbm, xv, yv, idxv, bv, sem):
        wid = lax.axis_index("s") * _NC + lax.axis_index("c")
        base = wid * per_w
        loads = [
            pltpu.async_copy(x_hbm.at[pl.ds(base, per_w)], xv, sem),
            pltpu.async_copy(y_hbm.at[pl.ds(base, per_w)], yv, sem),
        ]
        for c in loads:
            c.wait()
        nct = n // 128
        for k in range(n_chunks):
            for j in range(chunk // _LANES):
                sl = pl.ds(k * chunk + j * _LANES, _LANES)
                sl2 = pl.ds(j * _LANES, _LANES)
                xs = xv[sl]
                ys = yv[sl]
                idxv[k, sl2] = (
                    (((xs >> 3) * nct + (ys >> 7)) << 10)
                    + ((xs & 7) << 7) + (ys & 127))
        copies = [
            pltpu.async_copy(bflat_hbm.at[idxv.at[k]], bv.at[k], sem)
            for k in range(n_chunks)
        ]
        for c in copies:
            c.wait()
        for k in range(n_chunks):
            pltpu.sync_copy(bv.at[k],
                            out_hbm.at[pl.ds(base + k * chunk, chunk)])

    return sc_bgather


def _make_sc_combine(batch, n):
    per_w = batch // _NW
    chunk = 128
    n_chunks = per_w // chunk
    mesh = plsc.VectorSubcoreMesh(core_axis_name="c", subcore_axis_name="s")

    @functools.partial(
        pl.kernel,
        mesh=mesh,
        out_type=jax.ShapeDtypeStruct((batch,), jnp.float32),
        scratch_types=[
            pltpu.VMEM((per_w,), jnp.float32),
            pltpu.VMEM((n_chunks, chunk), jnp.int32),
            pltpu.VMEM((n_chunks, chunk), jnp.float32),
            pltpu.VMEM((per_w,), jnp.float32),
            pltpu.SemaphoreType.DMA,
            [pltpu.SemaphoreType.DMA] * 4,
            pltpu.SemaphoreType.DMA,
        ],
    )
    def sc_combine(y_hbm, b_hbm, g_hbm, out_hbm,
                   bvals, yidx, gv, outv, sem, gsems, osem):
        wid = lax.axis_index("s") * _NC + lax.axis_index("c")
        base = wid * per_w
        loads = []
        for k in range(n_chunks):
            loads.append(pltpu.async_copy(
                y_hbm.at[pl.ds(base + k * chunk, chunk)], yidx.at[k], sem))
        loads.append(
            pltpu.async_copy(b_hbm.at[pl.ds(base, per_w)], bvals, sem))
        for c in loads:
            c.wait()
        copies = [
            pltpu.async_copy(g_hbm.at[yidx.at[k]], gv.at[k], gsems[k % 4])
            for k in range(n_chunks)
        ]
        stores = []
        for k in range(n_chunks):
            copies[k].wait()
            for j in range(chunk // _LANES):
                sl = pl.ds(k * chunk + j * _LANES, _LANES)
                sl2 = pl.ds(j * _LANES, _LANES)
                b = bvals[sl]
                val = jnp.exp(b - gv[k, sl2])
                outv[sl] = jnp.where(b != 0.0, val, 0.0)
            stores.append(pltpu.async_copy(
                outv.at[pl.ds(k * chunk, chunk)],
                out_hbm.at[pl.ds(base + k * chunk, chunk)], osem))
        for c in stores:
            c.wait()

    return sc_combine


def kernel(x, y, B):
    n = B.shape[0]
    batch = x.shape[0]
    xf = x.reshape(batch)
    yf = y.reshape(batch)
    bt = jnp.transpose(
        B.reshape(n // 8, 8, n // 128, 128), (0, 2, 1, 3)).reshape(n * n)
    bvals = _make_sc_bgather(batch, n)(xf, yf, bt)
    g = _colstats(B)
    out = _make_sc_combine(batch, n)(yf, bvals, g)
    return out.reshape(x.shape)
